# Initial kernel scaffold; baseline (speedup 1.0000x reference)
import jax
import jax.numpy as jnp
from jax.experimental import pallas as pl


def kernel(pos, batch, w1a, b1a, w1b, b1b, w2a, b2a, w2b, b2b, w3a, b3a, w3b, b3b, wl1, bl1, wl2, bl2, wl3, bl3):
    raise NotImplementedError("write your pallas kernel here")


# speedup vs baseline: 15.3949x; 15.3949x over previous
import functools

import jax
import jax.numpy as jnp
from jax.experimental import pallas as pl

_B = 16
_K = 32

_INTERPRET = False


def _conv_body(*refs, r2, RB, Np, has_x):
    if has_x:
        p_ref, pT_ref, x_ref, wax_ref, wap_ref, ba_ref, wb_ref, bb_ref, out_ref = refs
    else:
        p_ref, pT_ref, wap_ref, ba_ref, wb_ref, bb_ref, out_ref = refs
    rb = pl.program_id(1)
    row0 = rb * RB

    p = p_ref[0]
    a = jnp.dot(p, wap_ref[...], preferred_element_type=jnp.float32)
    a = a + ba_ref[...]
    if has_x:
        a = a + jnp.dot(x_ref[0], wax_ref[...],
                        preferred_element_type=jnp.float32)
    p_rows = p_ref[0, pl.ds(row0, RB), :]
    c_rows = jnp.dot(p_rows, wap_ref[...],
                     preferred_element_type=jnp.float32)

    dx = p_ref[0, pl.ds(row0, RB), 0:1] - pT_ref[0, 0:1, :]
    dy = p_ref[0, pl.ds(row0, RB), 1:2] - pT_ref[0, 1:2, :]
    dz = p_ref[0, pl.ds(row0, RB), 2:3] - pT_ref[0, 2:3, :]
    dsq = (dx * dx + dy * dy) + dz * dz
    dsq = jnp.where(dsq <= r2, dsq, jnp.inf)

    iota = jax.lax.broadcasted_iota(jnp.int32, (RB, Np), 1)
    F = wb_ref.shape[1]
    out = jnp.full((RB, F), -jnp.inf, dtype=jnp.float32)
    for _ in range(_K):
        m = jnp.min(dsq, axis=1, keepdims=True)
        ji = jnp.min(jnp.where(dsq == m, iota, Np), axis=1,
                     keepdims=True)
        oh = (iota == ji).astype(jnp.float32)
        g = jnp.dot(oh, a, preferred_element_type=jnp.float32)
        h = jnp.dot(jnp.maximum(g - c_rows, 0.0), wb_ref[...],
                    preferred_element_type=jnp.float32) + bb_ref[...]
        h = jnp.where(m < jnp.inf, h, -jnp.inf)
        out = jnp.maximum(out, h)
        dsq = jnp.where(iota == ji, jnp.inf, dsq)
    out_ref[0] = jnp.maximum(out, 0.0)


def _conv_layer(p, pT, x, wax, wap, ba, wb, bb, r2, RB):
    Bb, Np, _ = p.shape
    H = wap.shape[1]
    F = wb.shape[1]
    nb = Np // RB
    has_x = x is not None
    ins = [p, pT]
    in_specs = [
        pl.BlockSpec((1, Np, 3), lambda b, r: (b, 0, 0)),
        pl.BlockSpec((1, 3, Np), lambda b, r: (b, 0, 0)),
    ]
    if has_x:
        ins += [x, wax]
        in_specs += [
            pl.BlockSpec((1, Np, x.shape[2]), lambda b, r: (b, 0, 0)),
            pl.BlockSpec(wax.shape, lambda b, r: (0, 0)),
        ]
    ins += [wap, ba, wb, bb]
    in_specs += [
        pl.BlockSpec((3, H), lambda b, r: (0, 0)),
        pl.BlockSpec((1, H), lambda b, r: (0, 0)),
        pl.BlockSpec((H, F), lambda b, r: (0, 0)),
        pl.BlockSpec((1, F), lambda b, r: (0, 0)),
    ]
    return pl.pallas_call(
        functools.partial(_conv_body, r2=r2, RB=RB, Np=Np, has_x=has_x),
        grid=(Bb, nb),
        in_specs=in_specs,
        out_specs=pl.BlockSpec((1, RB, F), lambda b, r: (b, r, 0)),
        out_shape=jax.ShapeDtypeStruct((Bb, Np, F), jnp.float32),
        interpret=_INTERPRET,
    )(*ins)


def _fps_body(pT_ref, sel_ref, *, M, Np, Bb):
    px = pT_ref[:, 0, :]
    py = pT_ref[:, 1, :]
    pz = pT_ref[:, 2, :]
    iota = jax.lax.broadcasted_iota(jnp.int32, (Bb, Np), 1)
    iota_m = jax.lax.broadcasted_iota(jnp.int32, (Bb, M), 1)

    def body(i, st):
        dist, last, selbuf = st
        oh = (iota == last).astype(jnp.float32)
        lx = jnp.sum(oh * px, axis=1, keepdims=True)
        ly = jnp.sum(oh * py, axis=1, keepdims=True)
        lz = jnp.sum(oh * pz, axis=1, keepdims=True)
        ddx = px - lx
        ddy = py - ly
        ddz = pz - lz
        d = (ddx * ddx + ddy * ddy) + ddz * ddz
        dist = jnp.minimum(dist, d)
        mx = jnp.max(dist, axis=1, keepdims=True)
        nxt = jnp.min(jnp.where(dist == mx, iota, Np), axis=1, keepdims=True)
        selbuf = jnp.where(iota_m == i, nxt, selbuf)
        return dist, nxt, selbuf

    _, _, selbuf = jax.lax.fori_loop(
        1, M, body,
        (jnp.full((Bb, Np), jnp.inf, dtype=jnp.float32),
         jnp.zeros((Bb, 1), dtype=jnp.int32),
         jnp.zeros((Bb, M), dtype=jnp.int32)))
    sel_ref[:, 0, :] = selbuf


def _fps(pT, M):
    Bb, _, Np = pT.shape
    return pl.pallas_call(
        functools.partial(_fps_body, M=M, Np=Np, Bb=Bb),
        out_shape=jax.ShapeDtypeStruct((Bb, 1, M), jnp.int32),
        interpret=_INTERPRET,
    )(pT)


def _gather_body(sel_ref, x_ref, p_ref, pT_ref, xg_ref, pg_ref, pgT_ref,
                 *, M, Np):
    sel = sel_ref[0]
    ohT = (jax.lax.broadcasted_iota(jnp.int32, (Np, M), 0)
           == sel).astype(jnp.float32)
    dn = (((0,), (0,)), ((), ()))
    xg_ref[0] = jax.lax.dot_general(ohT, x_ref[0], dn,
                                    preferred_element_type=jnp.float32)
    pg_ref[0] = jax.lax.dot_general(ohT, p_ref[0], dn,
                                    preferred_element_type=jnp.float32)
    pgT_ref[0] = jnp.dot(pT_ref[0], ohT,
                         preferred_element_type=jnp.float32)


def _gather(sel, x, p, pT):
    Bb, Np, F = x.shape
    M = sel.shape[2]
    return pl.pallas_call(
        functools.partial(_gather_body, M=M, Np=Np),
        grid=(Bb,),
        in_specs=[
            pl.BlockSpec((1, 1, M), lambda b: (b, 0, 0)),
            pl.BlockSpec((1, Np, F), lambda b: (b, 0, 0)),
            pl.BlockSpec((1, Np, 3), lambda b: (b, 0, 0)),
            pl.BlockSpec((1, 3, Np), lambda b: (b, 0, 0)),
        ],
        out_specs=[
            pl.BlockSpec((1, M, F), lambda b: (b, 0, 0)),
            pl.BlockSpec((1, M, 3), lambda b: (b, 0, 0)),
            pl.BlockSpec((1, 3, M), lambda b: (b, 0, 0)),
        ],
        out_shape=[
            jax.ShapeDtypeStruct((Bb, M, F), jnp.float32),
            jax.ShapeDtypeStruct((Bb, M, 3), jnp.float32),
            jax.ShapeDtypeStruct((Bb, 3, M), jnp.float32),
        ],
        interpret=_INTERPRET,
    )(sel, x, p, pT)


def _head_body(x_ref, w1_ref, b1_ref, w2_ref, b2_ref, w3_ref, b3_ref,
               out_ref):
    g = jnp.max(x_ref[...], axis=1)
    h = jnp.maximum(
        jnp.dot(g, w1_ref[...], preferred_element_type=jnp.float32)
        + b1_ref[...], 0.0)
    h = jnp.maximum(
        jnp.dot(h, w2_ref[...], preferred_element_type=jnp.float32)
        + b2_ref[...], 0.0)
    lo = jnp.dot(h, w3_ref[...], preferred_element_type=jnp.float32) \
        + b3_ref[...]
    s = lo - jnp.max(lo, axis=1, keepdims=True)
    out_ref[...] = s - jnp.log(jnp.sum(jnp.exp(s), axis=1, keepdims=True))


def _head(x, wl1, bl1, wl2, bl2, wl3, bl3):
    Bb = x.shape[0]
    NC = wl3.shape[1]
    return pl.pallas_call(
        _head_body,
        out_shape=jax.ShapeDtypeStruct((Bb, NC), jnp.float32),
        interpret=_INTERPRET,
    )(x, wl1, bl1, wl2, bl2, wl3, bl3)


def kernel(pos, batch, w1a, b1a, w1b, b1b, w2a, b2a, w2b, b2b,
           w3a, b3a, w3b, b3b, wl1, bl1, wl2, bl2, wl3, bl3):
    Np = pos.shape[0] // _B
    p0 = pos.reshape(_B, Np, 3)
    pT0 = p0.transpose(0, 2, 1)
    r1 = (1, -1)
    b1a_, b1b_, b2a_, b2b_, b3a_, b3b_ = (
        b.reshape(r1) for b in (b1a, b1b, b2a, b2b, b3a, b3b))
    bl1_, bl2_, bl3_ = (b.reshape(r1) for b in (bl1, bl2, bl3))

    RB1 = min(256, Np)
    x1 = _conv_layer(p0, pT0, None, None, w1a, b1a_, w1b, b1b_,
                     0.2 * 0.2, RB1)
    sel1 = _fps(pT0, Np // 2)
    x1g, p1, p1T = _gather(sel1, x1, p0, pT0)

    M1 = Np // 2
    RB2 = min(256, M1)
    x2 = _conv_layer(p1, p1T, x1g, w2a[:64], w2a[64:], b2a_, w2b, b2b_,
                     0.4 * 0.4, RB2)
    sel2 = _fps(p1T, Np // 8)
    x2g, p2, p2T = _gather(sel2, x2, p1, p1T)

    M2 = Np // 8
    RB3 = min(256, M2)
    x3 = _conv_layer(p2, p2T, x2g, w3a[:128], w3a[128:], b3a_, w3b, b3b_,
                     1.0 * 1.0, RB3)

    return _head(x3, wl1, bl1_, wl2, bl2_, wl3, bl3_)


# speedup vs baseline: 17.1056x; 1.1111x over previous
import functools

import jax
import jax.numpy as jnp
from jax.experimental import pallas as pl

_B = 16
_K = 32

_INTERPRET = False


def _conv_body(*refs, r2, RB, Np, has_x):
    if has_x:
        p_ref, pT_ref, x_ref, wax_ref, wap_ref, ba_ref, wb_ref, bb_ref, out_ref = refs
    else:
        p_ref, pT_ref, wap_ref, ba_ref, wb_ref, bb_ref, out_ref = refs
    rb = pl.program_id(1)
    row0 = rb * RB

    p = p_ref[0]
    a = jnp.dot(p, wap_ref[...], preferred_element_type=jnp.float32)
    a = a + ba_ref[...]
    if has_x:
        a = a + jnp.dot(x_ref[0], wax_ref[...],
                        preferred_element_type=jnp.float32)
    p_rows = p_ref[0, pl.ds(row0, RB), :]
    c_rows = jnp.dot(p_rows, wap_ref[...],
                     preferred_element_type=jnp.float32)

    dx = p_ref[0, pl.ds(row0, RB), 0:1] - pT_ref[0, 0:1, :]
    dy = p_ref[0, pl.ds(row0, RB), 1:2] - pT_ref[0, 1:2, :]
    dz = p_ref[0, pl.ds(row0, RB), 2:3] - pT_ref[0, 2:3, :]
    dsq = (dx * dx + dy * dy) + dz * dz
    dsq = jnp.where(dsq <= r2, dsq, jnp.inf)

    H = a.shape[1]
    a_hi = a.astype(jnp.bfloat16)
    a_lo = (a - a_hi.astype(jnp.float32)).astype(jnp.bfloat16)
    a_cat = jnp.concatenate([a_hi, a_lo], axis=1)

    iota = jax.lax.broadcasted_iota(jnp.int32, (RB, Np), 1)
    F = wb_ref.shape[1]
    out = jnp.full((RB, F), -jnp.inf, dtype=jnp.float32)
    for _ in range(_K):
        m = jnp.min(dsq, axis=1, keepdims=True)
        ji = jnp.min(jnp.where(dsq == m, iota, Np), axis=1,
                     keepdims=True)
        oh = (iota == ji).astype(jnp.bfloat16)
        g2 = jnp.dot(oh, a_cat, preferred_element_type=jnp.float32)
        g = g2[:, :H] + g2[:, H:]
        h = jnp.dot(jnp.maximum(g - c_rows, 0.0), wb_ref[...],
                    preferred_element_type=jnp.float32) + bb_ref[...]
        h = jnp.where(m < jnp.inf, h, -jnp.inf)
        out = jnp.maximum(out, h)
        dsq = jnp.where(iota == ji, jnp.inf, dsq)
    out_ref[0] = jnp.maximum(out, 0.0)


def _conv_layer(p, pT, x, wax, wap, ba, wb, bb, r2, RB):
    Bb, Np, _ = p.shape
    H = wap.shape[1]
    F = wb.shape[1]
    nb = Np // RB
    has_x = x is not None
    ins = [p, pT]
    in_specs = [
        pl.BlockSpec((1, Np, 3), lambda b, r: (b, 0, 0)),
        pl.BlockSpec((1, 3, Np), lambda b, r: (b, 0, 0)),
    ]
    if has_x:
        ins += [x, wax]
        in_specs += [
            pl.BlockSpec((1, Np, x.shape[2]), lambda b, r: (b, 0, 0)),
            pl.BlockSpec(wax.shape, lambda b, r: (0, 0)),
        ]
    ins += [wap, ba, wb, bb]
    in_specs += [
        pl.BlockSpec((3, H), lambda b, r: (0, 0)),
        pl.BlockSpec((1, H), lambda b, r: (0, 0)),
        pl.BlockSpec((H, F), lambda b, r: (0, 0)),
        pl.BlockSpec((1, F), lambda b, r: (0, 0)),
    ]
    return pl.pallas_call(
        functools.partial(_conv_body, r2=r2, RB=RB, Np=Np, has_x=has_x),
        grid=(Bb, nb),
        in_specs=in_specs,
        out_specs=pl.BlockSpec((1, RB, F), lambda b, r: (b, r, 0)),
        out_shape=jax.ShapeDtypeStruct((Bb, Np, F), jnp.float32),
        interpret=_INTERPRET,
    )(*ins)


def _fps_body(pT_ref, sel_ref, *, M, Np, Bb):
    px = pT_ref[:, 0, :]
    py = pT_ref[:, 1, :]
    pz = pT_ref[:, 2, :]
    iota = jax.lax.broadcasted_iota(jnp.int32, (Bb, Np), 1)
    iota_m = jax.lax.broadcasted_iota(jnp.int32, (Bb, M), 1)

    def body(i, st):
        dist, last, selbuf = st
        oh = (iota == last).astype(jnp.float32)
        lx = jnp.sum(oh * px, axis=1, keepdims=True)
        ly = jnp.sum(oh * py, axis=1, keepdims=True)
        lz = jnp.sum(oh * pz, axis=1, keepdims=True)
        ddx = px - lx
        ddy = py - ly
        ddz = pz - lz
        d = (ddx * ddx + ddy * ddy) + ddz * ddz
        dist = jnp.minimum(dist, d)
        mx = jnp.max(dist, axis=1, keepdims=True)
        nxt = jnp.min(jnp.where(dist == mx, iota, Np), axis=1, keepdims=True)
        selbuf = jnp.where(iota_m == i, nxt, selbuf)
        return dist, nxt, selbuf

    _, _, selbuf = jax.lax.fori_loop(
        1, M, body,
        (jnp.full((Bb, Np), jnp.inf, dtype=jnp.float32),
         jnp.zeros((Bb, 1), dtype=jnp.int32),
         jnp.zeros((Bb, M), dtype=jnp.int32)))
    sel_ref[:, 0, :] = selbuf


def _fps(pT, M):
    Bb, _, Np = pT.shape
    return pl.pallas_call(
        functools.partial(_fps_body, M=M, Np=Np, Bb=Bb),
        out_shape=jax.ShapeDtypeStruct((Bb, 1, M), jnp.int32),
        interpret=_INTERPRET,
    )(pT)


def _gather_body(sel_ref, x_ref, p_ref, pT_ref, xg_ref, pg_ref, pgT_ref,
                 *, M, Np):
    sel = sel_ref[0]
    ohT = (jax.lax.broadcasted_iota(jnp.int32, (Np, M), 0)
           == sel).astype(jnp.float32)
    dn = (((0,), (0,)), ((), ()))
    xg_ref[0] = jax.lax.dot_general(ohT, x_ref[0], dn,
                                    preferred_element_type=jnp.float32)
    pg_ref[0] = jax.lax.dot_general(ohT, p_ref[0], dn,
                                    preferred_element_type=jnp.float32)
    pgT_ref[0] = jnp.dot(pT_ref[0], ohT,
                         preferred_element_type=jnp.float32)


def _gather(sel, x, p, pT):
    Bb, Np, F = x.shape
    M = sel.shape[2]
    return pl.pallas_call(
        functools.partial(_gather_body, M=M, Np=Np),
        grid=(Bb,),
        in_specs=[
            pl.BlockSpec((1, 1, M), lambda b: (b, 0, 0)),
            pl.BlockSpec((1, Np, F), lambda b: (b, 0, 0)),
            pl.BlockSpec((1, Np, 3), lambda b: (b, 0, 0)),
            pl.BlockSpec((1, 3, Np), lambda b: (b, 0, 0)),
        ],
        out_specs=[
            pl.BlockSpec((1, M, F), lambda b: (b, 0, 0)),
            pl.BlockSpec((1, M, 3), lambda b: (b, 0, 0)),
            pl.BlockSpec((1, 3, M), lambda b: (b, 0, 0)),
        ],
        out_shape=[
            jax.ShapeDtypeStruct((Bb, M, F), jnp.float32),
            jax.ShapeDtypeStruct((Bb, M, 3), jnp.float32),
            jax.ShapeDtypeStruct((Bb, 3, M), jnp.float32),
        ],
        interpret=_INTERPRET,
    )(sel, x, p, pT)


def _head_body(x_ref, w1_ref, b1_ref, w2_ref, b2_ref, w3_ref, b3_ref,
               out_ref):
    g = jnp.max(x_ref[...], axis=1)
    h = jnp.maximum(
        jnp.dot(g, w1_ref[...], preferred_element_type=jnp.float32)
        + b1_ref[...], 0.0)
    h = jnp.maximum(
        jnp.dot(h, w2_ref[...], preferred_element_type=jnp.float32)
        + b2_ref[...], 0.0)
    lo = jnp.dot(h, w3_ref[...], preferred_element_type=jnp.float32) \
        + b3_ref[...]
    s = lo - jnp.max(lo, axis=1, keepdims=True)
    out_ref[...] = s - jnp.log(jnp.sum(jnp.exp(s), axis=1, keepdims=True))


def _head(x, wl1, bl1, wl2, bl2, wl3, bl3):
    Bb = x.shape[0]
    NC = wl3.shape[1]
    return pl.pallas_call(
        _head_body,
        out_shape=jax.ShapeDtypeStruct((Bb, NC), jnp.float32),
        interpret=_INTERPRET,
    )(x, wl1, bl1, wl2, bl2, wl3, bl3)


def kernel(pos, batch, w1a, b1a, w1b, b1b, w2a, b2a, w2b, b2b,
           w3a, b3a, w3b, b3b, wl1, bl1, wl2, bl2, wl3, bl3):
    Np = pos.shape[0] // _B
    p0 = pos.reshape(_B, Np, 3)
    pT0 = p0.transpose(0, 2, 1)
    r1 = (1, -1)
    b1a_, b1b_, b2a_, b2b_, b3a_, b3b_ = (
        b.reshape(r1) for b in (b1a, b1b, b2a, b2b, b3a, b3b))
    bl1_, bl2_, bl3_ = (b.reshape(r1) for b in (bl1, bl2, bl3))

    RB1 = min(256, Np)
    x1 = _conv_layer(p0, pT0, None, None, w1a, b1a_, w1b, b1b_,
                     0.2 * 0.2, RB1)
    sel1 = _fps(pT0, Np // 2)
    x1g, p1, p1T = _gather(sel1, x1, p0, pT0)

    M1 = Np // 2
    RB2 = min(256, M1)
    x2 = _conv_layer(p1, p1T, x1g, w2a[:64], w2a[64:], b2a_, w2b, b2b_,
                     0.4 * 0.4, RB2)
    sel2 = _fps(p1T, Np // 8)
    x2g, p2, p2T = _gather(sel2, x2, p1, p1T)

    M2 = Np // 8
    RB3 = min(256, M2)
    x3 = _conv_layer(p2, p2T, x2g, w3a[:128], w3a[128:], b3a_, w3b, b3b_,
                     1.0 * 1.0, RB3)

    return _head(x3, wl1, bl1_, wl2, bl2_, wl3, bl3_)


# speedup vs baseline: 17.1328x; 1.0016x over previous
import functools

import jax
import jax.numpy as jnp
from jax.experimental import pallas as pl

_B = 16
_K = 32

_INTERPRET = False


def _conv_body(*refs, r2, RB, Np, has_x):
    if has_x:
        p_ref, pT_ref, x_ref, wax_ref, wap_ref, ba_ref, wb_ref, bb_ref, out_ref = refs
    else:
        p_ref, pT_ref, wap_ref, ba_ref, wb_ref, bb_ref, out_ref = refs
    rb = pl.program_id(1)
    row0 = rb * RB

    p = p_ref[0]
    a = jnp.dot(p, wap_ref[...], preferred_element_type=jnp.float32)
    a = a + ba_ref[...]
    if has_x:
        a = a + jnp.dot(x_ref[0], wax_ref[...],
                        preferred_element_type=jnp.float32)
    p_rows = p_ref[0, pl.ds(row0, RB), :]
    c_rows = jnp.dot(p_rows, wap_ref[...],
                     preferred_element_type=jnp.float32)

    dx = p_ref[0, pl.ds(row0, RB), 0:1] - pT_ref[0, 0:1, :]
    dy = p_ref[0, pl.ds(row0, RB), 1:2] - pT_ref[0, 1:2, :]
    dz = p_ref[0, pl.ds(row0, RB), 2:3] - pT_ref[0, 2:3, :]
    dsq = (dx * dx + dy * dy) + dz * dz
    dsq = jnp.where(dsq <= r2, dsq, jnp.inf)

    H = a.shape[1]
    a_hi = a.astype(jnp.bfloat16)
    a_lo = (a - a_hi.astype(jnp.float32)).astype(jnp.bfloat16)
    a_cat = jnp.concatenate([a_hi, a_lo], axis=1)

    NCH = 4
    CH = RB // NCH
    iota_c = jax.lax.broadcasted_iota(jnp.int32, (CH, Np), 1)
    F = wb_ref.shape[1]
    out = jnp.full((RB, F), -jnp.inf, dtype=jnp.float32)
    chains = [dsq[i * CH:(i + 1) * CH, :] for i in range(NCH)]
    for _ in range(_K):
        ohs = []
        ms = []
        for i in range(NCH):
            d = chains[i]
            m = jnp.min(d, axis=1, keepdims=True)
            ji = jnp.min(jnp.where(d == m, iota_c, Np), axis=1,
                         keepdims=True)
            cmp = iota_c == ji
            chains[i] = jnp.where(cmp, jnp.inf, d)
            ohs.append(cmp.astype(jnp.bfloat16))
            ms.append(m)
        oh = jnp.concatenate(ohs, axis=0)
        m_all = jnp.concatenate(ms, axis=0)
        g2 = jnp.dot(oh, a_cat, preferred_element_type=jnp.float32)
        g = g2[:, :H] + g2[:, H:]
        h = jnp.dot(jnp.maximum(g - c_rows, 0.0), wb_ref[...],
                    preferred_element_type=jnp.float32) + bb_ref[...]
        h = jnp.where(m_all < jnp.inf, h, -jnp.inf)
        out = jnp.maximum(out, h)
    out_ref[0] = jnp.maximum(out, 0.0)


def _conv_layer(p, pT, x, wax, wap, ba, wb, bb, r2, RB):
    Bb, Np, _ = p.shape
    H = wap.shape[1]
    F = wb.shape[1]
    nb = Np // RB
    has_x = x is not None
    ins = [p, pT]
    in_specs = [
        pl.BlockSpec((1, Np, 3), lambda b, r: (b, 0, 0)),
        pl.BlockSpec((1, 3, Np), lambda b, r: (b, 0, 0)),
    ]
    if has_x:
        ins += [x, wax]
        in_specs += [
            pl.BlockSpec((1, Np, x.shape[2]), lambda b, r: (b, 0, 0)),
            pl.BlockSpec(wax.shape, lambda b, r: (0, 0)),
        ]
    ins += [wap, ba, wb, bb]
    in_specs += [
        pl.BlockSpec((3, H), lambda b, r: (0, 0)),
        pl.BlockSpec((1, H), lambda b, r: (0, 0)),
        pl.BlockSpec((H, F), lambda b, r: (0, 0)),
        pl.BlockSpec((1, F), lambda b, r: (0, 0)),
    ]
    return pl.pallas_call(
        functools.partial(_conv_body, r2=r2, RB=RB, Np=Np, has_x=has_x),
        grid=(Bb, nb),
        in_specs=in_specs,
        out_specs=pl.BlockSpec((1, RB, F), lambda b, r: (b, r, 0)),
        out_shape=jax.ShapeDtypeStruct((Bb, Np, F), jnp.float32),
        interpret=_INTERPRET,
    )(*ins)


def _fps_body(pT_ref, sel_ref, *, M, Np, Bb):
    px = pT_ref[:, 0, :]
    py = pT_ref[:, 1, :]
    pz = pT_ref[:, 2, :]
    iota = jax.lax.broadcasted_iota(jnp.int32, (Bb, Np), 1)
    iota_m = jax.lax.broadcasted_iota(jnp.int32, (Bb, M), 1)

    def body(i, st):
        dist, last, selbuf = st
        oh = (iota == last).astype(jnp.float32)
        lx = jnp.sum(oh * px, axis=1, keepdims=True)
        ly = jnp.sum(oh * py, axis=1, keepdims=True)
        lz = jnp.sum(oh * pz, axis=1, keepdims=True)
        ddx = px - lx
        ddy = py - ly
        ddz = pz - lz
        d = (ddx * ddx + ddy * ddy) + ddz * ddz
        dist = jnp.minimum(dist, d)
        mx = jnp.max(dist, axis=1, keepdims=True)
        nxt = jnp.min(jnp.where(dist == mx, iota, Np), axis=1, keepdims=True)
        selbuf = jnp.where(iota_m == i, nxt, selbuf)
        return dist, nxt, selbuf

    _, _, selbuf = jax.lax.fori_loop(
        1, M, body,
        (jnp.full((Bb, Np), jnp.inf, dtype=jnp.float32),
         jnp.zeros((Bb, 1), dtype=jnp.int32),
         jnp.zeros((Bb, M), dtype=jnp.int32)))
    sel_ref[:, 0, :] = selbuf


def _fps(pT, M):
    Bb, _, Np = pT.shape
    return pl.pallas_call(
        functools.partial(_fps_body, M=M, Np=Np, Bb=Bb),
        out_shape=jax.ShapeDtypeStruct((Bb, 1, M), jnp.int32),
        interpret=_INTERPRET,
    )(pT)


def _gather_body(sel_ref, x_ref, p_ref, pT_ref, xg_ref, pg_ref, pgT_ref,
                 *, M, Np):
    sel = sel_ref[0]
    ohT = (jax.lax.broadcasted_iota(jnp.int32, (Np, M), 0)
           == sel).astype(jnp.float32)
    dn = (((0,), (0,)), ((), ()))
    xg_ref[0] = jax.lax.dot_general(ohT, x_ref[0], dn,
                                    preferred_element_type=jnp.float32)
    pg_ref[0] = jax.lax.dot_general(ohT, p_ref[0], dn,
                                    preferred_element_type=jnp.float32)
    pgT_ref[0] = jnp.dot(pT_ref[0], ohT,
                         preferred_element_type=jnp.float32)


def _gather(sel, x, p, pT):
    Bb, Np, F = x.shape
    M = sel.shape[2]
    return pl.pallas_call(
        functools.partial(_gather_body, M=M, Np=Np),
        grid=(Bb,),
        in_specs=[
            pl.BlockSpec((1, 1, M), lambda b: (b, 0, 0)),
            pl.BlockSpec((1, Np, F), lambda b: (b, 0, 0)),
            pl.BlockSpec((1, Np, 3), lambda b: (b, 0, 0)),
            pl.BlockSpec((1, 3, Np), lambda b: (b, 0, 0)),
        ],
        out_specs=[
            pl.BlockSpec((1, M, F), lambda b: (b, 0, 0)),
            pl.BlockSpec((1, M, 3), lambda b: (b, 0, 0)),
            pl.BlockSpec((1, 3, M), lambda b: (b, 0, 0)),
        ],
        out_shape=[
            jax.ShapeDtypeStruct((Bb, M, F), jnp.float32),
            jax.ShapeDtypeStruct((Bb, M, 3), jnp.float32),
            jax.ShapeDtypeStruct((Bb, 3, M), jnp.float32),
        ],
        interpret=_INTERPRET,
    )(sel, x, p, pT)


def _head_body(x_ref, w1_ref, b1_ref, w2_ref, b2_ref, w3_ref, b3_ref,
               out_ref):
    g = jnp.max(x_ref[...], axis=1)
    h = jnp.maximum(
        jnp.dot(g, w1_ref[...], preferred_element_type=jnp.float32)
        + b1_ref[...], 0.0)
    h = jnp.maximum(
        jnp.dot(h, w2_ref[...], preferred_element_type=jnp.float32)
        + b2_ref[...], 0.0)
    lo = jnp.dot(h, w3_ref[...], preferred_element_type=jnp.float32) \
        + b3_ref[...]
    s = lo - jnp.max(lo, axis=1, keepdims=True)
    out_ref[...] = s - jnp.log(jnp.sum(jnp.exp(s), axis=1, keepdims=True))


def _head(x, wl1, bl1, wl2, bl2, wl3, bl3):
    Bb = x.shape[0]
    NC = wl3.shape[1]
    return pl.pallas_call(
        _head_body,
        out_shape=jax.ShapeDtypeStruct((Bb, NC), jnp.float32),
        interpret=_INTERPRET,
    )(x, wl1, bl1, wl2, bl2, wl3, bl3)


def kernel(pos, batch, w1a, b1a, w1b, b1b, w2a, b2a, w2b, b2b,
           w3a, b3a, w3b, b3b, wl1, bl1, wl2, bl2, wl3, bl3):
    Np = pos.shape[0] // _B
    p0 = pos.reshape(_B, Np, 3)
    pT0 = p0.transpose(0, 2, 1)
    r1 = (1, -1)
    b1a_, b1b_, b2a_, b2b_, b3a_, b3b_ = (
        b.reshape(r1) for b in (b1a, b1b, b2a, b2b, b3a, b3b))
    bl1_, bl2_, bl3_ = (b.reshape(r1) for b in (bl1, bl2, bl3))

    RB1 = min(256, Np)
    x1 = _conv_layer(p0, pT0, None, None, w1a, b1a_, w1b, b1b_,
                     0.2 * 0.2, RB1)
    sel1 = _fps(pT0, Np // 2)
    x1g, p1, p1T = _gather(sel1, x1, p0, pT0)

    M1 = Np // 2
    RB2 = min(256, M1)
    x2 = _conv_layer(p1, p1T, x1g, w2a[:64], w2a[64:], b2a_, w2b, b2b_,
                     0.4 * 0.4, RB2)
    sel2 = _fps(p1T, Np // 8)
    x2g, p2, p2T = _gather(sel2, x2, p1, p1T)

    M2 = Np // 8
    RB3 = min(256, M2)
    x3 = _conv_layer(p2, p2T, x2g, w3a[:128], w3a[128:], b3a_, w3b, b3b_,
                     1.0 * 1.0, RB3)

    return _head(x3, wl1, bl1_, wl2, bl2_, wl3, bl3_)
